# baseline (speedup 1.0000x reference)
import functools

import jax
import jax.numpy as jnp
from jax import lax
from jax.experimental import pallas as pl
from jax.experimental.pallas import tpu as pltpu
from jax.experimental.pallas import tpu_sc as plsc

N = 320000
D = 128
H = 16
NSEG = 10000

NC = 2
NS = 16
NW = NC * NS
IR = N // D
IR_PAD = 2504
IR_BASE = IR // NW
IR_EXTRA = IR % NW
MAX_IRPT = IR_BASE + 1
IDXBUF = 88
G = 1
NG = IR_BASE // G
ZU = (NSEG + D - 1) // D
ZTAIL = NSEG - (ZU - 1) * D


def _sc_segment_sum(x, batch2d):
    mesh = plsc.VectorSubcoreMesh(core_axis_name="c", subcore_axis_name="s")

    @functools.partial(
        pl.kernel,
        mesh=mesh,
        out_type=jax.ShapeDtypeStruct((NC, NSEG, D), jnp.float32),
        scratch_types=[
            pltpu.VMEM((2, G * D, D), jnp.float32),
            pltpu.VMEM((IDXBUF, D), jnp.int32),
            pltpu.VMEM((D,), jnp.int32),
            pltpu.VMEM_SHARED((NSEG, D), jnp.float32),
            pltpu.SemaphoreType.DMA,
            pltpu.SemaphoreType.DMA,
            pltpu.SemaphoreType.DMA,
            pltpu.SemaphoreType.DMA,
        ],
    )
    def seg_kernel(x_hbm, b_hbm, out_hbm, rows_v, idx_v, ident_v, acc_sh,
                   sem_g0, sem_g1, sem_s0, sem_s1):
        c = lax.axis_index("c")
        s = lax.axis_index("s")
        w = c * NS + s

        nck = IR_BASE + (w < IR_EXTRA).astype(jnp.int32)
        a = IR_BASE * w + jnp.minimum(w, IR_EXTRA)
        sa = (a // 8) * 8
        o = a - sa

        gsems = (sem_g0, sem_g1)
        ssems = (sem_s0, sem_s1)

        def prep_ident(k):
            if k % 3 == 2:
                rowbase = (a + k) * D
                for j in range(D // 16):
                    ident_v[pl.ds(16 * j, 16)] = (
                        lax.iota(jnp.int32, 16) + rowbase + 16 * j
                    )

        def gather(k, buf):
            if k % 3 == 2:
                return pltpu.make_async_copy(
                    x_hbm.at[ident_v], rows_v.at[buf], gsems[buf]
                )
            return pltpu.make_async_copy(
                x_hbm.at[pl.ds((a + k) * D, D)], rows_v.at[buf], gsems[buf]
            )

        def scatter(k, buf):
            return pltpu.make_async_copy(
                rows_v.at[buf], acc_sh.at[idx_v.at[o + k]], ssems[buf]
            )

        idx_fetch = pltpu.make_async_copy(
            b_hbm.at[pl.ds(sa, IDXBUF)], idx_v, sem_s1
        )
        idx_fetch.start()

        zeros16 = jnp.zeros((16,), jnp.float32)

        def zbody(i, carry):
            r = i // (D // 16)
            q = i % (D // 16)
            rows_v[0, r, pl.ds(q * 16, 16)] = zeros16
            return carry

        lax.fori_loop(0, D * (D // 16), zbody, 0)
        zcopies = []
        for k in range(5):
            u = s + NS * k

            @pl.when(u < ZU - 1)
            def _():
                pltpu.async_copy(
                    rows_v.at[0, pl.ds(0, D)],
                    acc_sh.at[pl.ds(u * D, D)],
                    sem_s0,
                )

            @pl.when(u == ZU - 1)
            def _():
                pltpu.async_copy(
                    rows_v.at[0, pl.ds(0, ZTAIL)],
                    acc_sh.at[pl.ds((ZU - 1) * D, ZTAIL)],
                    sem_s0,
                )
        gather(1, 1).start()
        for k in range(5):
            u = s + NS * k

            @pl.when(u < ZU - 1)
            def _():
                pltpu.make_async_copy(
                    rows_v.at[0, pl.ds(0, D)],
                    acc_sh.at[pl.ds(u * D, D)],
                    sem_s0,
                ).wait()

            @pl.when(u == ZU - 1)
            def _():
                pltpu.make_async_copy(
                    rows_v.at[0, pl.ds(0, ZTAIL)],
                    acc_sh.at[pl.ds((ZU - 1) * D, ZTAIL)],
                    sem_s0,
                ).wait()
        gather(0, 0).start()
        idx_fetch.wait()
        plsc.subcore_barrier()

        for k in range(NG):
            b = k % 2
            gather(k, b).wait()
            if k + 1 < NG and k >= 1:
                scatter(k - 1, 1 - b).wait()
                prep_ident(k + 1)
                gather(k + 1, 1 - b).start()
            scatter(k, b).start(add=True)
        scatter(NG - 2, (NG - 2) % 2).wait()
        scatter(NG - 1, (NG - 1) % 2).wait()

        @pl.when(nck == MAX_IRPT)
        def _():
            pltpu.sync_copy(
                x_hbm.at[pl.ds((a + IR_BASE) * D, D)],
                rows_v.at[0, pl.ds(0, D)],
            )
            pltpu.sync_copy(
                rows_v.at[0, pl.ds(0, D)],
                acc_sh.at[idx_v.at[o + IR_BASE]],
                add=True,
            )
        plsc.subcore_barrier()

        for wait in (False, True):
            for k in range(5):
                u = s + NS * k

                @pl.when(u < ZU - 1)
                def _():
                    cp = pltpu.make_async_copy(
                        acc_sh.at[pl.ds(u * D, D)],
                        out_hbm.at[c, pl.ds(u * D, D)],
                        sem_g0,
                    )
                    cp.wait() if wait else cp.start()

                @pl.when(u == ZU - 1)
                def _():
                    cp = pltpu.make_async_copy(
                        acc_sh.at[pl.ds((ZU - 1) * D, ZTAIL)],
                        out_hbm.at[c, pl.ds((ZU - 1) * D, ZTAIL)],
                        sem_g0,
                    )
                    cp.wait() if wait else cp.start()

    return seg_kernel(x, batch2d)


def _mlp(partials, W1, b1, W2, b2):
    def mlp_kernel(p_ref, w1_ref, b1_ref, w2_ref, b2_ref, o_ref):
        pooled = p_ref[0] + p_ref[1]
        h = jnp.dot(pooled, w1_ref[...], preferred_element_type=jnp.float32)
        h = jnp.maximum(h + b1_ref[...], 0.0)
        o = jnp.dot(h, w2_ref[...], preferred_element_type=jnp.float32)
        o_ref[...] = o + b2_ref[...]

    return pl.pallas_call(
        mlp_kernel,
        out_shape=jax.ShapeDtypeStruct((NSEG, 1), jnp.float32),
    )(partials, W1, b1.reshape(1, H), W2, b2.reshape(1, 1))


def kernel(x, batch, W1, b1, W2, b2):
    bi = batch.astype(jnp.int32)
    bi = jnp.pad(bi, (0, IR_PAD * D - N))
    batch2d = bi.reshape(IR_PAD, D)
    partials = _sc_segment_sum(x, batch2d)
    out = _mlp(partials, W1, b1, W2, b2)
    return out.reshape(NSEG)


# speedup vs baseline: 1.0038x; 1.0038x over previous
import functools

import jax
import jax.numpy as jnp
from jax import lax
from jax.experimental import pallas as pl
from jax.experimental.pallas import tpu as pltpu
from jax.experimental.pallas import tpu_sc as plsc

N = 320000
D = 128
H = 16
NSEG = 10000

NC = 2
NS = 16
NW = NC * NS
IR = N // D
IR_PAD = 2504
IR_BASE = IR // NW
IR_EXTRA = IR % NW
MAX_IRPT = IR_BASE + 1
IDXBUF = 88
G = 1
NG = IR_BASE // G
ZU = (NSEG + D - 1) // D
ZTAIL = NSEG - (ZU - 1) * D


def _sc_segment_sum(x, batch2d):
    mesh = plsc.VectorSubcoreMesh(core_axis_name="c", subcore_axis_name="s")

    @functools.partial(
        pl.kernel,
        mesh=mesh,
        out_type=jax.ShapeDtypeStruct((NC, NSEG, D), jnp.float32),
        scratch_types=[
            pltpu.VMEM((2, G * D, D), jnp.float32),
            pltpu.VMEM((IDXBUF, D), jnp.int32),
            pltpu.VMEM_SHARED((NSEG, D), jnp.float32),
            pltpu.SemaphoreType.DMA,
            pltpu.SemaphoreType.DMA,
            pltpu.SemaphoreType.DMA,
            pltpu.SemaphoreType.DMA,
        ],
    )
    def seg_kernel(x_hbm, b_hbm, out_hbm, rows_v, idx_v, acc_sh,
                   sem_g0, sem_g1, sem_s0, sem_s1):
        c = lax.axis_index("c")
        s = lax.axis_index("s")
        w = c * NS + s

        nck = IR_BASE + (w < IR_EXTRA).astype(jnp.int32)
        a = IR_BASE * w + jnp.minimum(w, IR_EXTRA)
        sa = (a // 8) * 8
        o = a - sa

        gsems = (sem_g0, sem_g1)
        ssems = (sem_s0, sem_s1)

        def gather(k, buf):
            return pltpu.make_async_copy(
                x_hbm.at[pl.ds((a + k) * D, D)], rows_v.at[buf], gsems[buf]
            )

        def scatter(k, buf):
            return pltpu.make_async_copy(
                rows_v.at[buf], acc_sh.at[idx_v.at[o + k]], ssems[buf]
            )

        idx_fetch = pltpu.make_async_copy(
            b_hbm.at[pl.ds(sa, IDXBUF)], idx_v, sem_s1
        )
        idx_fetch.start()

        zeros16 = jnp.zeros((16,), jnp.float32)

        def zbody(i, carry):
            r = i // (D // 16)
            q = i % (D // 16)
            rows_v[0, r, pl.ds(q * 16, 16)] = zeros16
            return carry

        lax.fori_loop(0, D * (D // 16), zbody, 0)
        for k in range(5):
            u = s + NS * k

            @pl.when(u < ZU - 1)
            def _():
                pltpu.async_copy(
                    rows_v.at[0, pl.ds(0, D)],
                    acc_sh.at[pl.ds(u * D, D)],
                    sem_s0,
                )

            @pl.when(u == ZU - 1)
            def _():
                pltpu.async_copy(
                    rows_v.at[0, pl.ds(0, ZTAIL)],
                    acc_sh.at[pl.ds((ZU - 1) * D, ZTAIL)],
                    sem_s0,
                )
        gather(1, 1).start()
        for k in range(5):
            u = s + NS * k

            @pl.when(u < ZU - 1)
            def _():
                pltpu.make_async_copy(
                    rows_v.at[0, pl.ds(0, D)],
                    acc_sh.at[pl.ds(u * D, D)],
                    sem_s0,
                ).wait()

            @pl.when(u == ZU - 1)
            def _():
                pltpu.make_async_copy(
                    rows_v.at[0, pl.ds(0, ZTAIL)],
                    acc_sh.at[pl.ds((ZU - 1) * D, ZTAIL)],
                    sem_s0,
                ).wait()
        gather(0, 0).start()
        idx_fetch.wait()
        plsc.subcore_barrier()

        for k in range(NG):
            b = k % 2
            gather(k, b).wait()
            if k + 1 < NG and k >= 1:
                scatter(k - 1, 1 - b).wait()
                gather(k + 1, 1 - b).start()
            scatter(k, b).start(add=True)
        scatter(NG - 2, (NG - 2) % 2).wait()
        scatter(NG - 1, (NG - 1) % 2).wait()

        @pl.when(nck == MAX_IRPT)
        def _():
            pltpu.sync_copy(
                x_hbm.at[pl.ds((a + IR_BASE) * D, D)],
                rows_v.at[0, pl.ds(0, D)],
            )
            pltpu.sync_copy(
                rows_v.at[0, pl.ds(0, D)],
                acc_sh.at[idx_v.at[o + IR_BASE]],
                add=True,
            )
        plsc.subcore_barrier()

        for wait in (False, True):
            for k in range(5):
                u = s + NS * k

                @pl.when(u < ZU - 1)
                def _():
                    cp = pltpu.make_async_copy(
                        acc_sh.at[pl.ds(u * D, D)],
                        out_hbm.at[c, pl.ds(u * D, D)],
                        sem_g0,
                    )
                    cp.wait() if wait else cp.start()

                @pl.when(u == ZU - 1)
                def _():
                    cp = pltpu.make_async_copy(
                        acc_sh.at[pl.ds((ZU - 1) * D, ZTAIL)],
                        out_hbm.at[c, pl.ds((ZU - 1) * D, ZTAIL)],
                        sem_g0,
                    )
                    cp.wait() if wait else cp.start()

    return seg_kernel(x, batch2d)


def _mlp(partials, W1, b1, W2, b2):
    def mlp_kernel(p_ref, w1_ref, b1_ref, w2_ref, b2_ref, o_ref):
        pooled = p_ref[0] + p_ref[1]
        h = jnp.dot(pooled, w1_ref[...], preferred_element_type=jnp.float32)
        h = jnp.maximum(h + b1_ref[...], 0.0)
        o = jnp.dot(h, w2_ref[...], preferred_element_type=jnp.float32)
        o_ref[...] = o + b2_ref[...]

    return pl.pallas_call(
        mlp_kernel,
        out_shape=jax.ShapeDtypeStruct((NSEG, 1), jnp.float32),
    )(partials, W1, b1.reshape(1, H), W2, b2.reshape(1, 1))


def kernel(x, batch, W1, b1, W2, b2):
    bi = batch.astype(jnp.int32)
    bi = jnp.pad(bi, (0, IR_PAD * D - N))
    batch2d = bi.reshape(IR_PAD, D)
    partials = _sc_segment_sum(x, batch2d)
    out = _mlp(partials, W1, b1, W2, b2)
    return out.reshape(NSEG)
